# baseline (speedup 1.0000x reference)
import functools

import jax
import jax.numpy as jnp
from jax import lax
from jax.experimental import pallas as pl
from jax.experimental.pallas import tpu as pltpu
from jax.experimental.pallas import tpu_sc as plsc

DIM = 64
SEN = 200
NC, NS = 2, 16
NW = NC * NS
BLK = 160
GS = 80
G = BLK // GS
PK = BLK // 16 * 8
NBUF = 2
NIB = 4


def _sc_embed(idx, W, pos, rows_total):
    blks = rows_total // (NW * BLK)

    @functools.partial(
        pl.kernel,
        out_type=jax.ShapeDtypeStruct((rows_total // 16 * 8, 128), jnp.float32),
        mesh=plsc.VectorSubcoreMesh(core_axis_name="c", subcore_axis_name="s"),
        scratch_types=[
            pltpu.VMEM((NIB, G, GS), jnp.int32),
            pltpu.VMEM((SEN, DIM), jnp.float32),
            pltpu.VMEM((NBUF, BLK, DIM), jnp.float32),
            pltpu.VMEM((NBUF, PK, 128), jnp.float32),
        ]
        + [pltpu.SemaphoreType.DMA] * (NIB + 2 * NBUF),
        compiler_params=pltpu.CompilerParams(use_tc_tiling_on_sc=False),
    )
    def k(idx_hbm, w_hbm, pos_hbm, out_hbm, idx_v, pos_v, rows_v, stg_v,
          *sems):
        isem = sems[:NIB]
        gsem = sems[NIB:NIB + NBUF]
        wsem = sems[NIB + NBUF:]
        wid = lax.axis_index("s") * NC + lax.axis_index("c")
        base_pk = wid * (blks * PK)
        pltpu.async_copy(pos_hbm, pos_v, gsem[0]).wait()

        def fetch_idx(blk, ib):
            pltpu.async_copy(idx_hbm.at[wid, blk], idx_v.at[ib], isem[ib])

        def wait_idx(ib):
            pltpu.make_async_copy(
                idx_hbm.at[wid, 0], idx_v.at[ib], isem[ib]
            ).wait()

        def start_gather(blk, ib, s):
            for g in range(G):
                pltpu.async_copy(
                    w_hbm.at[idx_v.at[ib, g]],
                    rows_v.at[s, pl.ds(g * GS, GS)],
                    gsem[s],
                )

        def wait_gather(s):
            pltpu.make_async_copy(
                w_hbm.at[pl.ds(0, BLK)], rows_v.at[s], gsem[s]
            ).wait()

        def wait_write(s):
            pltpu.make_async_copy(
                stg_v.at[s], out_hbm.at[pl.ds(0, PK)], wsem[s]
            ).wait()

        def _process(b, s, ib, u):
            wait_gather(s)

            @pl.when(b >= NBUF)
            def _():
                wait_write(s)

            phase = lax.rem(b * BLK, SEN)

            @pl.loop(0, BLK, step=16)
            def _(i0):
                for r16 in range(16):
                    i = i0 + r16
                    a, hb = r16 % 8, r16 // 8
                    p = phase + i
                    p = jnp.where(p >= SEN, p - SEN, p)
                    q8 = i0 // 2
                    for c in range(DIM // 16):
                        stg_v[s, q8 + a, pl.ds(hb * DIM + c * 16, 16)] = (
                            rows_v[s, i, pl.ds(c * 16, 16)]
                            + pos_v[p, pl.ds(c * 16, 16)]
                        )

            pltpu.async_copy(
                stg_v.at[s],
                out_hbm.at[pl.ds(base_pk + b * PK, PK)],
                wsem[s],
            )

            nxt = b + 1

            @pl.when(nxt < blks)
            def _():
                wait_idx((u + 1) % NIB)
                start_gather(nxt, (u + 1) % NIB, (u + 1) % NBUF)

            pf = b + 3

            @pl.when(pf < blks)
            def _():
                fetch_idx(pf, (u + 3) % NIB)

        for b in range(min(3, blks)):
            fetch_idx(b, b % NIB)
        wait_idx(0)
        start_gather(0, 0, 0)

        @pl.loop(0, blks, step=NIB)
        def _(b0):
            for u in range(NIB):
                _process(b0 + u, u % NBUF, u, u)

        for s in range(NBUF):
            wait_write(s)

    return k(idx, W, pos)


def kernel(inputs, W, pos_table):
    B, S = inputs.shape
    rows_total = B * S
    blks = rows_total // (NW * BLK)
    idx = inputs.reshape(NW, blks, G, GS)
    pos = pos_table[1 : S + 1]
    out = _sc_embed(idx, W, pos, rows_total)
    out = out.reshape(rows_total // 16, 8, 2, DIM)
    out = jnp.transpose(out, (0, 2, 1, 3))
    return out.reshape(B, S, DIM)
